# baseline (speedup 1.0000x reference)
import functools

import jax
import jax.numpy as jnp
from jax import lax
from jax.experimental import pallas as pl
from jax.experimental.pallas import tpu as pltpu
from jax.experimental.pallas import tpu_sc as plsc

B, L, V, D = 4096, 50, 1000, 32
NT = 4
P = B * L
NW = 32
PPT = P // NW
GCH = 256
NCH = PPT // GCH
NG = NCH * NT
NB = 4
NITER = NG // NB

_mesh = plsc.VectorSubcoreMesh(core_axis_name="c", subcore_axis_name="s")


@functools.partial(
    pl.kernel,
    mesh=_mesh,
    out_type=jax.ShapeDtypeStruct((P, NT * D), jnp.float32),
    scratch_types=(
        [pltpu.VMEM((PPT,), jnp.int32) for _ in range(NT)]
        + [pltpu.VMEM((NB, GCH, D), jnp.float32)]
        + [pltpu.SemaphoreType.DMA for _ in range(2 * NB)]
    ),
    compiler_params=pltpu.CompilerParams(use_tc_tiling_on_sc=False),
)
def _seq_gather(t0_hbm, t1_hbm, t2_hbm, t3_hbm,
                i0_hbm, i1_hbm, i2_hbm, i3_hbm, out_hbm,
                s0, s1, s2, s3, rows, *sems):
    gsem = sems[:NB]
    osem = sems[NB:]
    idx_bufs = (s0, s1, s2, s3)
    idx_hbms = (i0_hbm, i1_hbm, i2_hbm, i3_hbm)
    tables = (t0_hbm, t1_hbm, t2_hbm, t3_hbm)

    wid = lax.axis_index("s") * 2 + lax.axis_index("c")
    pos_base = wid * PPT

    for t in range(NT):
        pltpu.sync_copy(idx_hbms[t].at[pl.ds(pos_base, PPT)], idx_bufs[t])

    def _gather(g_dyn, b, t):
        idx = idx_bufs[t].at[pl.ds((g_dyn // NT) * GCH, GCH)]
        return pltpu.make_async_copy(tables[t].at[idx], rows.at[b], gsem[b])

    def _writeout(g_dyn, b, t):
        dst = out_hbm.at[pl.ds(pos_base + (g_dyn // NT) * GCH, GCH),
                         pl.ds(t * D, D)]
        return pltpu.make_async_copy(rows.at[b], dst, osem[b])

    for b in range(NB):
        _gather(b, b, b % NT).start()

    def _step(i, carry):
        g0 = i * NB
        for b in range(NB):
            t = b % NT
            g = g0 + b
            _gather(g, b, t).wait()
            _writeout(g, b, t).start()

            @pl.when(i < NITER - 1)
            def _():
                _writeout(g, b, t).wait()
                _gather(g + NB, b, t).start()
        return carry

    lax.fori_loop(0, NITER, _step, 0)

    for b in range(NB):
        _writeout(NG - NB + b, b, b % NT).wait()


def _ctx_body(lr_ref, mc_ref, nc_ref, pt_ref, out_ref):
    col = lax.broadcasted_iota(jnp.int32, (B, 35), 1)
    onehot = ((col - 3) == pt_ref[...]).astype(jnp.int32)
    vi = jnp.where(col == 0, lr_ref[...],
                   jnp.where(col == 1, mc_ref[...],
                             jnp.where(col == 2, nc_ref[...], onehot)))
    out_ref[...] = vi.astype(jnp.float32)


def kernel(bwd_vectors, fwd_vectors, last_bwd_vectors, last_fwd_vectors,
           last_rule, move_counter, node_count, problem_type,
           emb_bwd, emb_fwd, emb_last_bwd, emb_last_fwd):
    seq_flat = _seq_gather(
        emb_bwd, emb_fwd, emb_last_bwd, emb_last_fwd,
        bwd_vectors.T.reshape(-1), fwd_vectors.T.reshape(-1),
        last_bwd_vectors.T.reshape(-1), last_fwd_vectors.T.reshape(-1))
    seq = seq_flat.reshape(L, B, NT * D).transpose(1, 0, 2)

    context = pl.pallas_call(
        _ctx_body,
        out_shape=jax.ShapeDtypeStruct((B, 35), jnp.float32),
    )(last_rule, move_counter, node_count, problem_type)

    sequence_length = jnp.full((B,), L, dtype=jnp.int32)
    return (context, seq, sequence_length)


# speedup vs baseline: 1.9772x; 1.9772x over previous
import functools

import jax
import jax.numpy as jnp
from jax import lax
from jax.experimental import pallas as pl
from jax.experimental.pallas import tpu as pltpu
from jax.experimental.pallas import tpu_sc as plsc

B, L, V, D = 4096, 50, 1000, 32
NT = 4
P = B * L
NW = 32
PPT = P // NW
GCH = 128
NCH = PPT // GCH
NG = NCH * NT
NB = 8
NITER = NG // NB

_mesh = plsc.VectorSubcoreMesh(core_axis_name="c", subcore_axis_name="s")


@functools.partial(
    pl.kernel,
    mesh=_mesh,
    out_type=jax.ShapeDtypeStruct((P, NT * D), jnp.float32),
    scratch_types=(
        [pltpu.VMEM((PPT,), jnp.int32) for _ in range(NT)]
        + [pltpu.VMEM((NB, GCH, D), jnp.float32),
           pltpu.VMEM_SHARED((NT, V, D), jnp.float32)]
        + [pltpu.SemaphoreType.DMA for _ in range(2 * NB)]
    ),
    compiler_params=pltpu.CompilerParams(use_tc_tiling_on_sc=False),
)
def _seq_gather(t0_hbm, t1_hbm, t2_hbm, t3_hbm,
                i0_hbm, i1_hbm, i2_hbm, i3_hbm, out_hbm,
                s0, s1, s2, s3, rows, shared_tab, *sems):
    gsem = sems[:NB]
    osem = sems[NB:]
    idx_bufs = (s0, s1, s2, s3)
    idx_hbms = (i0_hbm, i1_hbm, i2_hbm, i3_hbm)
    tables = (t0_hbm, t1_hbm, t2_hbm, t3_hbm)

    wid = lax.axis_index("s") * 2 + lax.axis_index("c")
    pos_base = wid * PPT

    @pl.when(lax.axis_index("s") == 0)
    def _():
        for t in range(NT):
            pltpu.sync_copy(tables[t], shared_tab.at[t])

    for t in range(NT):
        pltpu.sync_copy(idx_hbms[t].at[pl.ds(pos_base, PPT)], idx_bufs[t])
    plsc.subcore_barrier()

    def _gather(g_dyn, b, t):
        idx = idx_bufs[t].at[pl.ds((g_dyn // NT) * GCH, GCH)]
        return pltpu.make_async_copy(shared_tab.at[t].at[idx], rows.at[b],
                                     gsem[b])

    def _writeout(g_dyn, b, t):
        dst = out_hbm.at[pl.ds(pos_base + (g_dyn // NT) * GCH, GCH),
                         pl.ds(t * D, D)]
        return pltpu.make_async_copy(rows.at[b], dst, osem[b])

    for b in range(NB):
        _gather(b, b, b % NT).start()

    def _step(i, carry):
        g0 = i * NB
        for b in range(NB):
            t = b % NT
            g = g0 + b
            _gather(g, b, t).wait()
            _writeout(g, b, t).start()

            @pl.when(i < NITER - 1)
            def _():
                _writeout(g, b, t).wait()
                _gather(g + NB, b, t).start()
        return carry

    lax.fori_loop(0, NITER, _step, 0)

    for b in range(NB):
        _writeout(NG - NB + b, b, b % NT).wait()


def _ctx_body(lr_ref, mc_ref, nc_ref, pt_ref, out_ref):
    col = lax.broadcasted_iota(jnp.int32, (B, 35), 1)
    onehot = ((col - 3) == pt_ref[...]).astype(jnp.int32)
    vi = jnp.where(col == 0, lr_ref[...],
                   jnp.where(col == 1, mc_ref[...],
                             jnp.where(col == 2, nc_ref[...], onehot)))
    out_ref[...] = vi.astype(jnp.float32)


def kernel(bwd_vectors, fwd_vectors, last_bwd_vectors, last_fwd_vectors,
           last_rule, move_counter, node_count, problem_type,
           emb_bwd, emb_fwd, emb_last_bwd, emb_last_fwd):
    seq_flat = _seq_gather(
        emb_bwd, emb_fwd, emb_last_bwd, emb_last_fwd,
        bwd_vectors.T.reshape(-1), fwd_vectors.T.reshape(-1),
        last_bwd_vectors.T.reshape(-1), last_fwd_vectors.T.reshape(-1))
    seq = seq_flat.reshape(L, B, NT * D).transpose(1, 0, 2)

    context = pl.pallas_call(
        _ctx_body,
        out_shape=jax.ShapeDtypeStruct((B, 35), jnp.float32),
    )(last_rule, move_counter, node_count, problem_type)

    sequence_length = jnp.full((B,), L, dtype=jnp.int32)
    return (context, seq, sequence_length)


# speedup vs baseline: 1.9994x; 1.0112x over previous
import functools

import jax
import jax.numpy as jnp
from jax import lax
from jax.experimental import pallas as pl
from jax.experimental.pallas import tpu as pltpu
from jax.experimental.pallas import tpu_sc as plsc

B, L, V, D = 4096, 50, 1000, 32
NT = 4
P = B * L
NW = 32
PPT = P // NW
GCH = 128
NCH = PPT // GCH
NG = NCH * NT
NB = 8
NITER = NG // NB

_mesh = plsc.VectorSubcoreMesh(core_axis_name="c", subcore_axis_name="s")


@functools.partial(
    pl.kernel,
    mesh=_mesh,
    out_type=jax.ShapeDtypeStruct((P, NT * D), jnp.float32),
    scratch_types=(
        [pltpu.VMEM((PPT,), jnp.int32) for _ in range(NT)]
        + [pltpu.VMEM((NB, GCH, D), jnp.float32),
           pltpu.VMEM_SHARED((NT, V, D), jnp.float32)]
        + [pltpu.SemaphoreType.DMA for _ in range(2 * NB)]
    ),
    compiler_params=pltpu.CompilerParams(use_tc_tiling_on_sc=False),
)
def _seq_gather(tab4_hbm, idx4_hbm, out_hbm,
                s0, s1, s2, s3, rows, shared_tab, *sems):
    gsem = sems[:NB]
    osem = sems[NB:]
    idx_bufs = (s0, s1, s2, s3)

    wid = lax.axis_index("s") * 2 + lax.axis_index("c")
    pos_base = wid * PPT

    @pl.when(lax.axis_index("s") == 0)
    def _():
        pltpu.sync_copy(tab4_hbm, shared_tab)

    for t in range(NT):
        pltpu.sync_copy(idx4_hbm.at[t].at[pl.ds(pos_base, PPT)], idx_bufs[t])
    plsc.subcore_barrier()

    def _gather(g_dyn, b, t):
        idx = idx_bufs[t].at[pl.ds((g_dyn // NT) * GCH, GCH)]
        return pltpu.make_async_copy(shared_tab.at[t].at[idx], rows.at[b],
                                     gsem[b])

    def _writeout(g_dyn, b, t):
        dst = out_hbm.at[pl.ds(pos_base + (g_dyn // NT) * GCH, GCH),
                         pl.ds(t * D, D)]
        return pltpu.make_async_copy(rows.at[b], dst, osem[b])

    for b in range(NB):
        _gather(b, b, b % NT).start()

    def _step(i, carry):
        g0 = i * NB
        for b in range(NB):
            t = b % NT
            g = g0 + b
            _gather(g, b, t).wait()
            _writeout(g, b, t).start()

            @pl.when(i < NITER - 1)
            def _():
                _writeout(g, b, t).wait()
                _gather(g + NB, b, t).start()
        return carry

    lax.fori_loop(0, NITER, _step, 0)

    for b in range(NB):
        _writeout(NG - NB + b, b, b % NT).wait()


def _ctx_body(lr_ref, mc_ref, nc_ref, pt_ref, out_ref):
    col = lax.broadcasted_iota(jnp.int32, (B, 35), 1)
    onehot = ((col - 3) == pt_ref[...]).astype(jnp.int32)
    vi = jnp.where(col == 0, lr_ref[...],
                   jnp.where(col == 1, mc_ref[...],
                             jnp.where(col == 2, nc_ref[...], onehot)))
    out_ref[...] = vi.astype(jnp.float32)


def kernel(bwd_vectors, fwd_vectors, last_bwd_vectors, last_fwd_vectors,
           last_rule, move_counter, node_count, problem_type,
           emb_bwd, emb_fwd, emb_last_bwd, emb_last_fwd):
    idx4 = jnp.stack([bwd_vectors.T.reshape(-1), fwd_vectors.T.reshape(-1),
                      last_bwd_vectors.T.reshape(-1),
                      last_fwd_vectors.T.reshape(-1)])
    tab4 = jnp.stack([emb_bwd, emb_fwd, emb_last_bwd, emb_last_fwd])
    seq_flat = _seq_gather(tab4, idx4)
    seq = seq_flat.reshape(L, B, NT * D).transpose(1, 0, 2)

    context = pl.pallas_call(
        _ctx_body,
        out_shape=jax.ShapeDtypeStruct((B, 35), jnp.float32),
    )(last_rule, move_counter, node_count, problem_type)

    sequence_length = jnp.full((B,), L, dtype=jnp.int32)
    return (context, seq, sequence_length)


# speedup vs baseline: 2.0625x; 1.0316x over previous
import functools

import jax
import jax.numpy as jnp
from jax import lax
from jax.experimental import pallas as pl
from jax.experimental.pallas import tpu as pltpu
from jax.experimental.pallas import tpu_sc as plsc

B, L, V, D = 4096, 50, 1000, 32
NT = 4
P = B * L
NW = 32
PPT = P // NW
GCH = 128
NCH = PPT // GCH
NG = NCH * NT
NB = 8
NITER = NG // NB

_mesh = plsc.VectorSubcoreMesh(core_axis_name="c", subcore_axis_name="s")


@functools.partial(
    pl.kernel,
    mesh=_mesh,
    out_type=jax.ShapeDtypeStruct((P, NT * D), jnp.float32),
    scratch_types=(
        [pltpu.VMEM((PPT,), jnp.int32) for _ in range(NT)]
        + [pltpu.VMEM((NB, GCH, D), jnp.float32),
           pltpu.VMEM_SHARED((NT, V, D), jnp.float32)]
        + [pltpu.SemaphoreType.DMA for _ in range(2 * NB)]
    ),
    compiler_params=pltpu.CompilerParams(use_tc_tiling_on_sc=False),
)
def _seq_gather(tab4_hbm, idx4_hbm, out_hbm,
                s0, s1, s2, s3, rows, shared_tab, *sems):
    gsem = sems[:NB]
    osem = sems[NB:]
    idx_bufs = (s0, s1, s2, s3)

    wid = lax.axis_index("s") * 2 + lax.axis_index("c")
    pos_base = wid * PPT

    @pl.when(lax.axis_index("s") == 0)
    def _():
        pltpu.sync_copy(tab4_hbm, shared_tab)

    for t in range(NT):
        pltpu.async_copy(idx4_hbm.at[t].at[pl.ds(pos_base, PPT)],
                         idx_bufs[t], gsem[t])
    plsc.subcore_barrier()
    for t in range(NT):
        pltpu.make_async_copy(idx4_hbm.at[t].at[pl.ds(pos_base, PPT)],
                              idx_bufs[t], gsem[t]).wait()

    def _gather(g_dyn, b, t):
        idx = idx_bufs[t].at[pl.ds((g_dyn // NT) * GCH, GCH)]
        return pltpu.make_async_copy(shared_tab.at[t].at[idx], rows.at[b],
                                     gsem[b])

    def _writeout(g_dyn, b, t):
        dst = out_hbm.at[pl.ds(pos_base + (g_dyn // NT) * GCH, GCH),
                         pl.ds(t * D, D)]
        return pltpu.make_async_copy(rows.at[b], dst, osem[b])

    for b in range(NB):
        _gather(b, b, b % NT).start()

    def _step(i, carry):
        g0 = i * NB
        for b in range(NB):
            t = b % NT
            g = g0 + b
            _gather(g, b, t).wait()
            _writeout(g, b, t).start()

            @pl.when(i < NITER - 1)
            def _():
                _writeout(g, b, t).wait()
                _gather(g + NB, b, t).start()
        return carry

    lax.fori_loop(0, NITER, _step, 0)

    for b in range(NB):
        _writeout(NG - NB + b, b, b % NT).wait()


def _ctx_body(lr_ref, mc_ref, nc_ref, pt_ref, out_ref):
    col = lax.broadcasted_iota(jnp.int32, (B, 35), 1)
    onehot = ((col - 3) == pt_ref[...]).astype(jnp.int32)
    vi = jnp.where(col == 0, lr_ref[...],
                   jnp.where(col == 1, mc_ref[...],
                             jnp.where(col == 2, nc_ref[...], onehot)))
    out_ref[...] = vi.astype(jnp.float32)


def kernel(bwd_vectors, fwd_vectors, last_bwd_vectors, last_fwd_vectors,
           last_rule, move_counter, node_count, problem_type,
           emb_bwd, emb_fwd, emb_last_bwd, emb_last_fwd):
    idx4 = jnp.stack([bwd_vectors.T.reshape(-1), fwd_vectors.T.reshape(-1),
                      last_bwd_vectors.T.reshape(-1),
                      last_fwd_vectors.T.reshape(-1)])
    tab4 = jnp.stack([emb_bwd, emb_fwd, emb_last_bwd, emb_last_fwd])
    seq_flat = _seq_gather(tab4, idx4)
    seq = seq_flat.reshape(L, B, NT * D).transpose(1, 0, 2)

    context = pl.pallas_call(
        _ctx_body,
        out_shape=jax.ShapeDtypeStruct((B, 35), jnp.float32),
    )(last_rule, move_counter, node_count, problem_type)

    sequence_length = jnp.full((B,), L, dtype=jnp.int32)
    return (context, seq, sequence_length)
